# Initial kernel scaffold; baseline (speedup 1.0000x reference)
import jax
import jax.numpy as jnp
from jax.experimental import pallas as pl


def kernel(x):
    raise NotImplementedError("write your pallas kernel here")


# speedup vs baseline: 2.7726x; 2.7726x over previous
import functools

import jax
import jax.numpy as jnp
from jax import lax
from jax.experimental import pallas as pl
from jax.experimental.pallas import tpu as pltpu
from jax.experimental.pallas import tpu_sc as plsc

_ROWS = 128
_N = 32768
_K = 64
_L = 16
_NVEC = _N // _L
_NC = 2
_NS = 16
_NW = _NC * _NS
_RPW = _ROWS // _NW


def _keyify(v):
    u = lax.bitcast_convert_type(v, jnp.int32)
    return jnp.where(u >= 0, u, u ^ jnp.int32(0x7FFFFFFF))


def _scan_hist(hist, kneed):
    lanes = lax.iota(jnp.int32, _L)

    def chunk(c, carry):
        acc, found, bstar, kp = carry
        base = (15 - c) * _L
        v = hist[pl.ds(base, _L)]
        for l in range(1, _L):
            v = v + hist[pl.ds(l * 256 + base, _L)]
        cs = jnp.cumsum(jnp.flip(v, 0))
        ge = acc + cs
        i0 = jnp.sum((ge < kneed).astype(jnp.int32))
        hit = i0 < _L
        v_at = jnp.sum(jnp.where(lanes == (15 - i0), v, 0))
        cs_at = jnp.sum(jnp.where(lanes == i0, cs, 0))
        upd = jnp.logical_and(hit, found == 0)
        bstar = jnp.where(upd, base + 15 - i0, bstar)
        kp = jnp.where(upd, kneed - (acc + cs_at - v_at), kp)
        found = jnp.where(hit, jnp.int32(1), found)
        acc = acc + jnp.sum(v)
        return acc, found, bstar, kp

    init = (jnp.int32(0), jnp.int32(0), jnp.int32(0), jnp.int32(0))
    _, _, bstar, kp = lax.fori_loop(0, _L, chunk, init)
    return bstar, kp


def _sc_body(x_hbm, out_hbm, buf, cand, hist):
    lanes = lax.iota(jnp.int32, _L)
    ones = jnp.ones((_L,), jnp.int32)
    wid = lax.axis_index("s") * _NC + lax.axis_index("c")

    def clr(i, c):
        hist[pl.ds(i * _L, _L)] = jnp.zeros((_L,), jnp.int32)
        return c

    def do_row(r, carry):
        row = wid * _RPW + r
        pltpu.sync_copy(x_hbm.at[row], buf)

        lax.fori_loop(0, 256, clr, 0)

        def p_hist3(i, c):
            v = buf[pl.ds(i * _L, _L)]
            k = _keyify(v)
            buf[pl.ds(i * _L, _L)] = lax.bitcast_convert_type(k, jnp.float32)
            b = (k >> 24) + 128
            plsc.addupdate_scatter(hist, [lanes * 256 + b], ones)
            return c

        lax.fori_loop(0, _NVEC, p_hist3, 0)
        b3, kneed = _scan_hist(hist, jnp.int32(_K))
        b3s = b3 - 128

        def p_compact3(i, off):
            k = lax.bitcast_convert_type(buf[pl.ds(i * _L, _L)], jnp.int32)
            msk = (k >> 24) == b3s
            plsc.store_compressed(cand.at[pl.ds(off, _L)], k, mask=msk)
            return off + jnp.sum(msk.astype(jnp.int32))

        m = lax.fori_loop(0, _NVEC, p_compact3, jnp.int32(0))

        def level(shift, m, kneed, compact):
            lax.fori_loop(0, 256, clr, 0)
            nv = (m + _L - 1) // _L

            def p_hist(i, c):
                k = cand[pl.ds(i * _L, _L)]
                valid = (i * _L + lanes) < m
                b = (k >> shift) & 255
                plsc.addupdate_scatter(hist, [lanes * 256 + b], ones, mask=valid)
                return c

            lax.fori_loop(0, nv, p_hist, 0)
            bs, kneed = _scan_hist(hist, kneed)
            if not compact:
                return bs, m, kneed

            def p_compact(i, off):
                k = cand[pl.ds(i * _L, _L)]
                valid = (i * _L + lanes) < m
                msk = jnp.logical_and(valid, ((k >> shift) & 255) == bs)
                plsc.store_compressed(cand.at[pl.ds(off, _L)], k, mask=msk)
                return off + jnp.sum(msk.astype(jnp.int32))

            m2 = lax.fori_loop(0, nv, p_compact, jnp.int32(0))
            return bs, m2, kneed

        b2, m, kneed = level(16, m, kneed, True)
        b1, m, kneed = level(8, m, kneed, True)
        b0, _, kneed = level(0, m, kneed, False)
        t = ((b3s * 256 + b2) * 256 + b1) * 256 + b0
        mfin = kneed

        def p_final(i, eq_seen):
            k = lax.bitcast_convert_type(buf[pl.ds(i * _L, _L)], jnp.int32)
            gt = k > t
            eq = k == t
            eqc = jnp.cumsum(eq.astype(jnp.int32))
            sel = jnp.logical_or(gt, jnp.logical_and(eq, (eq_seen + eqc) <= mfin))
            outv = jnp.where(jnp.logical_and(sel, k > 0),
                             lax.bitcast_convert_type(k, jnp.float32),
                             jnp.float32(0))
            buf[pl.ds(i * _L, _L)] = outv
            return eq_seen + jnp.sum(eq.astype(jnp.int32))

        lax.fori_loop(0, _NVEC, p_final, jnp.int32(0))
        pltpu.sync_copy(buf, out_hbm.at[row])
        return carry

    lax.fori_loop(0, _RPW, do_row, 0)


@jax.jit
def kernel(x):
    mesh = plsc.VectorSubcoreMesh(core_axis_name="c", subcore_axis_name="s")
    run = pl.kernel(
        _sc_body,
        out_type=jax.ShapeDtypeStruct((_ROWS, _N), jnp.float32),
        mesh=mesh,
        scratch_types=[
            pltpu.VMEM((_N,), jnp.float32),
            pltpu.VMEM((_N + _L,), jnp.int32),
            pltpu.VMEM((_L * 256,), jnp.int32),
        ],
        compiler_params=pltpu.CompilerParams(needs_layout_passes=False),
    )
    return run(x)


# speedup vs baseline: 8.9466x; 3.2268x over previous
import jax
import jax.numpy as jnp
from jax import lax
from jax.experimental import pallas as pl
from jax.experimental.pallas import tpu as pltpu
from jax.experimental.pallas import tpu_sc as plsc

_ROWS = 128
_N = 32768
_K = 64
_L = 16
_NVEC = _N // _L
_NC = 2
_NS = 16
_NW = _NC * _NS
_RPW = _ROWS // _NW


def _keyify(v):
    u = lax.bitcast_convert_type(v, jnp.int32)
    return jnp.where(u >= 0, u, u ^ jnp.int32(0x7FFFFFFF))


def _popcount(mask):
    r = plsc.all_reduce_population_count(mask)
    return r[0] if r.ndim else r


def _scan_hist(hist, kneed):
    lanes = lax.iota(jnp.int32, _L)
    zeros = jnp.zeros((_L,), jnp.int32)

    def chunk(c, carry):
        acc, found, bstar, kp = carry
        base = (15 - c) * _L
        v = hist[pl.ds(base, _L)]
        hist[pl.ds(base, _L)] = zeros
        for l in range(1, _L):
            s = l * 256 + base
            v = v + hist[pl.ds(s, _L)]
            hist[pl.ds(s, _L)] = zeros
        cs = jnp.cumsum(jnp.flip(v, 0))
        i0 = _popcount(acc + cs < kneed)
        hit = i0 < _L
        cs_prev = jnp.sum(jnp.where(lanes == i0 - 1, cs, 0))
        upd = jnp.logical_and(hit, found == 0)
        bstar = jnp.where(upd, base + _L - 1 - i0, bstar)
        kp = jnp.where(upd, kneed - acc - cs_prev, kp)
        found = jnp.where(hit, jnp.int32(1), found)
        acc = acc + cs[_L - 1]
        return acc, found, bstar, kp

    init = (jnp.int32(0), jnp.int32(0), jnp.int32(0), jnp.int32(0))
    _, _, bstar, kp = lax.fori_loop(0, _L, chunk, init)
    return bstar, kp


def _sc_body(x_hbm, out_hbm, buf0, buf1, cand, hist, si0, si1, so0, so1):
    lanes = lax.iota(jnp.int32, _L)
    ones = jnp.ones((_L,), jnp.int32)
    laneoff = lanes * 256
    wid = lax.axis_index("s") * _NC + lax.axis_index("c")
    row0 = wid * _RPW

    def clr(i, c):
        hist[pl.ds(i * _L, _L)] = jnp.zeros((_L,), jnp.int32)
        return c
    lax.fori_loop(0, 256, clr, 0)

    def process(buf, row):
        @plsc.parallel_loop(0, _NVEC, unroll=8)
        def p_hist3(i):
            k = _keyify(buf[pl.ds(i * _L, _L)])
            plsc.addupdate_scatter(hist, [laneoff + (k >> 24) + 128], ones)

        b3, kneed = _scan_hist(hist, jnp.int32(_K))
        b3s = b3 - 128

        @plsc.parallel_loop(0, _NVEC, unroll=4, carry=jnp.int32(0))
        def p_compact3(i, off):
            k = _keyify(buf[pl.ds(i * _L, _L)])
            msk = (k >> 24) == b3s
            plsc.store_compressed(cand.at[pl.ds(off, _L)], i * _L + lanes, mask=msk)
            return off + _popcount(msk)

        m = p_compact3

        def level(shift, m, kneed):
            nv = (m + _L - 1) // _L

            def p_hist(i, c):
                pos = cand[pl.ds(i * _L, _L)]
                valid = (i * _L + lanes) < m
                k = _keyify(plsc.load_gather(buf, [pos], mask=valid))
                b = (k >> shift) & 255
                plsc.addupdate_scatter(hist, [laneoff + b], ones, mask=valid)
                return c

            lax.fori_loop(0, nv, p_hist, 0)
            bs, kneed = _scan_hist(hist, kneed)

            def p_compact(i, off):
                pos = cand[pl.ds(i * _L, _L)]
                valid = (i * _L + lanes) < m
                k = _keyify(plsc.load_gather(buf, [pos], mask=valid))
                msk = jnp.logical_and(valid, ((k >> shift) & 255) == bs)
                plsc.store_compressed(cand.at[pl.ds(off, _L)], pos, mask=msk)
                return off + _popcount(msk)

            m2 = lax.fori_loop(0, nv, p_compact, jnp.int32(0))
            return bs, m2, kneed

        b2, m, kneed = level(16, m, kneed)
        b1, m, kneed = level(8, m, kneed)
        b0, m, kneed = level(0, m, kneed)
        t = ((b3s * 256 + b2) * 256 + b1) * 256 + b0
        mfin = kneed
        tmax = jnp.maximum(t, jnp.int32(0))

        @plsc.parallel_loop(0, _NVEC, unroll=8)
        def p_final(i):
            k = _keyify(buf[pl.ds(i * _L, _L)])
            keep = k > tmax
            buf[pl.ds(i * _L, _L)] = jnp.where(
                keep, lax.bitcast_convert_type(k, jnp.float32), jnp.float32(0))

        tf = jnp.broadcast_to(lax.bitcast_convert_type(t, jnp.float32), (_L,))
        nvt = (mfin + _L - 1) // _L

        def p_tie(i, c):
            pos = cand[pl.ds(i * _L, _L)]
            msk = jnp.logical_and((i * _L + lanes) < mfin, t > 0)
            plsc.store_scatter(buf, [pos], tf, mask=msk)
            return c

        lax.fori_loop(0, nvt, p_tie, 0)

    bufs = (buf0, buf1)
    sin = (si0, si1)
    sout = (so0, so1)
    in_h = [None] * _RPW
    out_h = [None] * _RPW
    in_h[0] = pltpu.async_copy(x_hbm.at[row0], buf0, si0)
    for r in range(_RPW):
        b = bufs[r % 2]
        if r + 1 < _RPW:
            if r >= 1:
                out_h[r - 1].wait()
            in_h[r + 1] = pltpu.async_copy(
                x_hbm.at[row0 + r + 1], bufs[(r + 1) % 2], sin[(r + 1) % 2])
        in_h[r].wait()
        process(b, row0 + r)
        out_h[r] = pltpu.async_copy(b, out_hbm.at[row0 + r], sout[r % 2])
    out_h[_RPW - 2].wait()
    out_h[_RPW - 1].wait()


@jax.jit
def kernel(x):
    mesh = plsc.VectorSubcoreMesh(core_axis_name="c", subcore_axis_name="s")
    run = pl.kernel(
        _sc_body,
        out_type=jax.ShapeDtypeStruct((_ROWS, _N), jnp.float32),
        mesh=mesh,
        scratch_types=[
            pltpu.VMEM((_N,), jnp.float32),
            pltpu.VMEM((_N,), jnp.float32),
            pltpu.VMEM((_N + _L,), jnp.int32),
            pltpu.VMEM((_L * 256,), jnp.int32),
            pltpu.SemaphoreType.DMA,
            pltpu.SemaphoreType.DMA,
            pltpu.SemaphoreType.DMA,
            pltpu.SemaphoreType.DMA,
        ],
        compiler_params=pltpu.CompilerParams(needs_layout_passes=False),
    )
    return run(x)
